# Initial kernel scaffold; baseline (speedup 1.0000x reference)
import jax
import jax.numpy as jnp
from jax.experimental import pallas as pl


def kernel(relative_position_bias_table, seq_len):
    raise NotImplementedError("write your pallas kernel here")


# speedup vs baseline: 87.7561x; 87.7561x over previous
import jax
import jax.numpy as jnp
from jax.experimental import pallas as pl
from jax.experimental.pallas import tpu as pltpu

BLOCK_ROWS = 256


def _toeplitz_body(r_ref, o_ref, scratch_ref):
    n = o_ref.shape[3]
    rb = pl.program_id(1)
    i0 = rb * BLOCK_ROWS

    @pl.when(rb == 0)
    def _build():
        row = r_ref[0]
        for s in range(128):
            shift = (s // 8) * 8 + (7 - s % 8)
            scratch_ref[s, :] = pltpu.roll(row, (2 * n - shift) % (2 * n), 1)[0]

    s_base = n - BLOCK_ROWS - i0
    for g in range(BLOCK_ROWS // 8):
        di0 = 8 * g
        o_top = BLOCK_ROWS - 1 - di0
        a = o_top // 128
        fbase = (BLOCK_ROWS - 8 - di0) - 128 * a
        base = pl.multiple_of(s_base + 128 * a, 128)
        o_ref[0, 0, pl.ds(di0, 8), :] = scratch_ref[pl.ds(fbase, 8),
                                                    pl.ds(base, n)]


def kernel(relative_position_bias_table, seq_len):
    table = relative_position_bias_table
    h = table.shape[1]
    n = (table.shape[0] + 1) // 2
    r = jnp.flip(table, axis=0).T
    r = jnp.pad(r, ((0, 0), (0, 1))).reshape(h, 1, 2 * n)

    out = pl.pallas_call(
        _toeplitz_body,
        grid=(h, n // BLOCK_ROWS),
        in_specs=[pl.BlockSpec((1, 1, 2 * n), lambda hh, rb: (hh, 0, 0))],
        out_specs=pl.BlockSpec((1, 1, BLOCK_ROWS, n),
                               lambda hh, rb: (0, hh, rb, 0)),
        out_shape=jax.ShapeDtypeStruct((1, h, n, n), table.dtype),
        scratch_shapes=[pltpu.VMEM((128, 2 * n), table.dtype)],
        compiler_params=pltpu.CompilerParams(
            dimension_semantics=("arbitrary", "arbitrary")),
    )(r)
    return out


# speedup vs baseline: 96.5983x; 1.1008x over previous
import jax
import jax.numpy as jnp
from jax.experimental import pallas as pl
from jax.experimental.pallas import tpu as pltpu

BLOCK_ROWS = 256


def _toeplitz_body(r_ref, o_ref, scratch_ref):
    n = o_ref.shape[3]
    two_n = r_ref.shape[1]
    hh = pl.program_id(0)
    rb = pl.program_id(1)
    i0 = rb * BLOCK_ROWS

    @pl.when(jnp.logical_and(hh == 0, rb == 0))
    def _build():
        rows = r_ref[...]
        for s in range(128):
            shift = (s // 8) * 8 + (7 - s % 8)
            scratch_ref[s, :, :] = pltpu.roll(rows, (two_n - shift) % two_n, 1)

    s_base = n - BLOCK_ROWS - i0
    for g in range(BLOCK_ROWS // 8):
        di0 = 8 * g
        o_top = BLOCK_ROWS - 1 - di0
        a = o_top // 128
        fbase = (BLOCK_ROWS - 8 - di0) - 128 * a
        base = pl.multiple_of(s_base + 128 * a, 128)
        o_ref[0, 0, pl.ds(di0, 8), :] = scratch_ref[pl.ds(fbase, 8), hh,
                                                    pl.ds(base, n)]


def kernel(relative_position_bias_table, seq_len):
    table = relative_position_bias_table
    h = table.shape[1]
    n = (table.shape[0] + 1) // 2
    r = jnp.flip(table, axis=0).T
    r = jnp.pad(r, ((0, 0), (0, 1)))

    out = pl.pallas_call(
        _toeplitz_body,
        grid=(h, n // BLOCK_ROWS),
        in_specs=[pl.BlockSpec((h, 2 * n), lambda hh, rb: (0, 0))],
        out_specs=pl.BlockSpec((1, 1, BLOCK_ROWS, n),
                               lambda hh, rb: (0, hh, rb, 0)),
        out_shape=jax.ShapeDtypeStruct((1, h, n, n), table.dtype),
        scratch_shapes=[pltpu.VMEM((128, h, 2 * n), table.dtype)],
        compiler_params=pltpu.CompilerParams(
            dimension_semantics=("arbitrary", "arbitrary")),
    )(r)
    return out
